# baseline (speedup 1.0000x reference)
import functools

import jax
import jax.numpy as jnp
from jax import lax
from jax.experimental import pallas as pl
from jax.experimental.pallas import tpu as pltpu
from jax.experimental.pallas import tpu_sc as plsc

_NC, _NS = 2, 16
_NW = _NC * _NS
_ROWS_PW = 312
_TAIL_BASE = _ROWS_PW * _NW
_TAIL = 10000 - _TAIL_BASE


def _sc_copy_body(x_hbm, b_hbm, xo_hbm, bo_hbm, xbuf, bbuf, sem):
    c = lax.axis_index("c")
    s = lax.axis_index("s")
    wid = s * _NC + c
    base = wid * _ROWS_PW
    sl = pl.ds(base, _ROWS_PW)
    bsl = pl.ds(0, _ROWS_PW)
    pltpu.async_copy(x_hbm.at[sl, :], xbuf.at[bsl, :], sem).wait()
    pltpu.async_copy(xbuf.at[bsl, :], xo_hbm.at[sl, :], sem).wait()

    @pl.when(wid == _NW - 1)
    def _():
        tsl = pl.ds(_TAIL_BASE, _TAIL)
        tbsl = pl.ds(0, _TAIL)
        pltpu.async_copy(x_hbm.at[tsl, :], xbuf.at[tbsl, :], sem).wait()
        pltpu.async_copy(xbuf.at[tbsl, :], xo_hbm.at[tsl, :], sem).wait()

    @pl.when(wid == 0)
    def _():
        pltpu.async_copy(b_hbm, bbuf, sem).wait()
        pltpu.async_copy(bbuf, bo_hbm, sem).wait()


def kernel(x, edge_index, batch):
    del edge_index
    mesh = plsc.VectorSubcoreMesh(core_axis_name="c", subcore_axis_name="s")
    sc_copy = functools.partial(
        pl.kernel,
        mesh=mesh,
        out_type=(
            jax.ShapeDtypeStruct(x.shape, x.dtype),
            jax.ShapeDtypeStruct(batch.shape, batch.dtype),
        ),
        scratch_types=[
            pltpu.VMEM((_ROWS_PW + _TAIL, x.shape[1]), x.dtype),
            pltpu.VMEM(batch.shape, batch.dtype),
            pltpu.SemaphoreType.DMA,
        ],
    )(_sc_copy_body)
    return sc_copy(x, batch)


# speedup vs baseline: 5.2393x; 5.2393x over previous
import jax
import jax.numpy as jnp
from jax.experimental import pallas as pl

_GRID = 2


def _copy_body(x_ref, b_ref, xo_ref, bo_ref):
    xo_ref[...] = x_ref[...]
    bo_ref[...] = b_ref[...]


def kernel(x, edge_index, batch):
    del edge_index
    n, d = x.shape
    rows = n // _GRID
    xo, bo = pl.pallas_call(
        _copy_body,
        grid=(_GRID,),
        in_specs=[
            pl.BlockSpec((rows, d), lambda i: (i, 0)),
            pl.BlockSpec(batch.shape, lambda i: (0,)),
        ],
        out_specs=(
            pl.BlockSpec((rows, d), lambda i: (i, 0)),
            pl.BlockSpec(batch.shape, lambda i: (0,)),
        ),
        out_shape=(
            jax.ShapeDtypeStruct(x.shape, x.dtype),
            jax.ShapeDtypeStruct(batch.shape, batch.dtype),
        ),
    )(x, batch)
    return (xo, bo)
